# baseline (speedup 1.0000x reference)
import functools

import jax
import jax.numpy as jnp
from jax import lax
from jax.experimental import pallas as pl
from jax.experimental.pallas import tpu as pltpu
from jax.experimental.pallas import tpu_sc as plsc

B = 64
N = 1024
K = 512
D = 96
N1 = N + 1
OUT_ROWS = K + 1

_R = 8
_JC = 128

_NP = 1032

_NC = 2
_NS = 16
_NW = _NC * _NS
_ROWS_PER_W = B // _NW

_PAD_SLOTS = 640
_CHUNK = 64
_NCHUNK = _PAD_SLOTS // _CHUNK

_OUT_PAD = 520


def _rank_body(rand_ref, x_ref, rank_ref, x128_ref):
    v = rand_ref[...]
    u = lax.bitcast_convert_type(v + 0.0, jnp.int32)
    k = u ^ ((u >> 31) & jnp.int32(0x7FFFFFFF))
    ka = k[:, :, None]

    ii = lax.broadcasted_iota(jnp.int32, (N, _JC), 0)
    jj0 = lax.broadcasted_iota(jnp.int32, (N, _JC), 1)

    acc3 = jnp.zeros((_R, N, _JC), jnp.int32)
    for jc in range(N // _JC):
        kb = k[:, None, jc * _JC:(jc + 1) * _JC]
        jlt = ((jc * _JC + jj0) < ii).astype(jnp.int32)
        acc3 = acc3 + ((kb + jlt[None]) > ka).astype(jnp.int32)
    rank = jnp.sum(acc3, axis=2)
    rank_ref[...] = rank.reshape(_R * (N // _JC), _JC)

    x128_ref[:, :N1, :D] = x_ref[...]


@jax.jit
def _ranks_tc(rand, x):
    return pl.pallas_call(
        _rank_body,
        grid=(B // _R,),
        in_specs=[
            pl.BlockSpec((_R, N), lambda g: (g, 0)),
            pl.BlockSpec((_R, N1, D), lambda g: (g, 0, 0)),
        ],
        out_specs=[
            pl.BlockSpec((_R * (N // _JC), _JC), lambda g: (g, 0)),
            pl.BlockSpec((_R, _NP, 128), lambda g: (g, 0, 0)),
        ],
        out_shape=[
            jax.ShapeDtypeStruct((B * (N // _JC), _JC), jnp.int32),
            jax.ShapeDtypeStruct((B, _NP, 128), jnp.float32),
        ],
    )(rand, x)


def _sc_body(x_hbm, ranks_hbm, out_hbm, rank_v, keep_v, rows_v, sem):
    wid = lax.axis_index("s") * _NC + lax.axis_index("c")
    iota = lax.iota(jnp.int32, 16)

    def row_body(t, carry):
        b = wid * _ROWS_PER_W + t
        pltpu.sync_copy(ranks_hbm.at[pl.ds(b * N, N)], rank_v)

        def zinit(g, c):
            p = g * 16 + iota
            val = jnp.where(p == 0, b * _NP, 0)
            plsc.store_scatter(keep_v, [p >> 6, p & 63], val)
            return c
        lax.fori_loop(0, _PAD_SLOTS // 16, zinit, 0)

        base_val = b * _NP + 1

        def scat(g, c):
            r = rank_v[pl.ds(g * 16, 16)]
            pos = jnp.minimum(r + 1, _PAD_SLOTS - 1)
            val = base_val + g * 16 + iota
            plsc.store_scatter(keep_v, [pos >> 6, pos & 63], val,
                               mask=r < K)
            return c
        lax.fori_loop(0, N // 16, scat, 0)

        copies = [
            pltpu.async_copy(
                x_hbm.at[keep_v.at[c]],
                rows_v.at[pl.ds(c * _CHUNK, _CHUNK)],
                sem,
            )
            for c in range(_NCHUNK)
        ]
        for cp in copies:
            cp.wait()
        pltpu.sync_copy(rows_v.at[pl.ds(0, _OUT_PAD)], out_hbm.at[b])
        return carry

    lax.fori_loop(0, _ROWS_PER_W, row_body, 0)


@jax.jit
def _gather_sc(x128, ranks_flat):
    mesh = plsc.VectorSubcoreMesh(core_axis_name="c", subcore_axis_name="s")
    run = functools.partial(
        pl.kernel,
        mesh=mesh,
        out_type=jax.ShapeDtypeStruct((B, _OUT_PAD, 128), jnp.float32),
        scratch_types=[
            pltpu.VMEM((N,), jnp.int32),
            pltpu.VMEM((_NCHUNK, _CHUNK), jnp.int32),
            pltpu.VMEM((_PAD_SLOTS, 128), jnp.float32),
            pltpu.SemaphoreType.DMA,
        ],
        compiler_params=pltpu.CompilerParams(
            needs_layout_passes=False, use_tc_tiling_on_sc=False),
    )(_sc_body)
    return run(x128, ranks_flat)


def kernel(x, rand):
    ranks, x128 = _ranks_tc(rand, x)
    padded = _gather_sc(x128.reshape(B * _NP, 128), ranks.reshape(-1))
    return padded[:, :OUT_ROWS, :D]


# speedup vs baseline: 3.5169x; 3.5169x over previous
import functools

import jax
import jax.numpy as jnp
from jax import lax
from jax.experimental import pallas as pl
from jax.experimental.pallas import tpu as pltpu
from jax.experimental.pallas import tpu_sc as plsc

B = 64
N = 1024
K = 512
D = 96
N1 = N + 1
OUT_ROWS = K + 1

_R = 8
_JC = 128

_NP = 1032

_NC = 2
_NS = 16
_NW = _NC * _NS
_ROWS_PER_W = B // _NW

_PAD_SLOTS = 640
_CHUNK = 64
_NCHUNK = _PAD_SLOTS // _CHUNK

_OUT_PAD = 520


def _rank_body(rand_ref, x_ref, rank_ref, x128_ref):
    v = rand_ref[...]
    u = lax.bitcast_convert_type(v + 0.0, jnp.int32)
    k = u ^ ((u >> 31) & jnp.int32(0x7FFFFFFF))
    ka = k[:, :, None]

    ii = lax.broadcasted_iota(jnp.int32, (N, _JC), 0)
    jj0 = lax.broadcasted_iota(jnp.int32, (N, _JC), 1)

    acc3 = jnp.zeros((_R, N, _JC), jnp.int32)
    for jc in range(N // _JC):
        kb = k[:, None, jc * _JC:(jc + 1) * _JC]
        jlt = ((jc * _JC + jj0) < ii).astype(jnp.int32)
        acc3 = acc3 + ((kb + jlt[None]) > ka).astype(jnp.int32)
    rank = jnp.sum(acc3, axis=2)
    rank_ref[...] = rank.reshape(_R * (N // _JC), _JC)

    x128_ref[:, :N1, :D] = x_ref[...]


@jax.jit
def _ranks_tc(rand, x):
    return pl.pallas_call(
        _rank_body,
        grid=(B // _R,),
        in_specs=[
            pl.BlockSpec((_R, N), lambda g: (g, 0)),
            pl.BlockSpec((_R, N1, D), lambda g: (g, 0, 0)),
        ],
        out_specs=[
            pl.BlockSpec((_R * (N // _JC), _JC), lambda g: (g, 0)),
            pl.BlockSpec((_R, _NP, 128), lambda g: (g, 0, 0)),
        ],
        out_shape=[
            jax.ShapeDtypeStruct((B * (N // _JC), _JC), jnp.int32),
            jax.ShapeDtypeStruct((B, _NP, 128), jnp.float32),
        ],
    )(rand, x)


def _sc_body(x_hbm, ranks_hbm, out_hbm, rank_v, keep_v, rows_v, sem):
    wid = lax.axis_index("s") * _NC + lax.axis_index("c")
    iota = lax.iota(jnp.int32, 16)

    def row_body(t, carry):
        b = wid * _ROWS_PER_W + t
        pltpu.sync_copy(ranks_hbm.at[pl.ds(b * N, N)], rank_v)

        def zinit(g, c):
            p = g * 16 + iota
            val = b * _NP + p
            plsc.store_scatter(keep_v, [p >> 6, p & 63], val)
            return c
        lax.fori_loop(0, _PAD_SLOTS // 16, zinit, 0)

        base_val = b * _NP + 1

        def scat(g, c):
            r = rank_v[pl.ds(g * 16, 16)]
            pos = jnp.minimum(r + 1, _PAD_SLOTS - 1)
            val = base_val + g * 16 + iota
            plsc.store_scatter(keep_v, [pos >> 6, pos & 63], val,
                               mask=r < K)
            return c
        lax.fori_loop(0, N // 16, scat, 0)

        copies = [
            pltpu.async_copy(
                x_hbm.at[keep_v.at[c]],
                rows_v.at[pl.ds(c * _CHUNK, _CHUNK)],
                sem,
            )
            for c in range(_NCHUNK)
        ]
        for cp in copies:
            cp.wait()
        pltpu.sync_copy(rows_v.at[pl.ds(0, _OUT_PAD)], out_hbm.at[b])
        return carry

    lax.fori_loop(0, _ROWS_PER_W, row_body, 0)


@jax.jit
def _gather_sc(x128, ranks_flat):
    mesh = plsc.VectorSubcoreMesh(core_axis_name="c", subcore_axis_name="s")
    run = functools.partial(
        pl.kernel,
        mesh=mesh,
        out_type=jax.ShapeDtypeStruct((B, _OUT_PAD, 128), jnp.float32),
        scratch_types=[
            pltpu.VMEM((N,), jnp.int32),
            pltpu.VMEM((_NCHUNK, _CHUNK), jnp.int32),
            pltpu.VMEM((_PAD_SLOTS, 128), jnp.float32),
            pltpu.SemaphoreType.DMA,
        ],
        compiler_params=pltpu.CompilerParams(
            needs_layout_passes=False, use_tc_tiling_on_sc=False),
    )(_sc_body)
    return run(x128, ranks_flat)


def kernel(x, rand):
    ranks, x128 = _ranks_tc(rand, x)
    padded = _gather_sc(x128.reshape(B * _NP, 128), ranks.reshape(-1))
    return padded[:, :OUT_ROWS, :D]
